# baseline (speedup 1.0000x reference)
import jax
import jax.numpy as jnp
from jax.experimental import pallas as pl


def _body(f_ref, m_ref, t_ref, o_ref):
    o_ref[...] = jnp.where(m_ref[...], t_ref[...], f_ref[...])


def kernel(features, mask, mask_token):
    B, S, D = features.shape
    N = B * S
    R = 2048
    f2 = features.reshape(N, D)
    m2 = mask.reshape(N, 1)
    t2 = mask_token.reshape(1, D)
    grid = (N // R,)
    out = pl.pallas_call(
        _body,
        grid=grid,
        in_specs=[
            pl.BlockSpec((R, D), lambda i: (i, 0)),
            pl.BlockSpec((R, 1), lambda i: (i, 0)),
            pl.BlockSpec((1, D), lambda i: (0, 0)),
        ],
        out_specs=pl.BlockSpec((R, D), lambda i: (i, 0)),
        out_shape=jax.ShapeDtypeStruct((N, D), features.dtype),
    )(f2, m2, t2)
    return out.reshape(B, S, D)
